# Initial kernel scaffold; baseline (speedup 1.0000x reference)
import jax
import jax.numpy as jnp
from jax.experimental import pallas as pl


def kernel(x, edge_index, edge_attr, batch, W1, b1, g1, be1, W2, b2, g2, be2, fW1, fb1, fW2, fb2, fW3, fb3):
    raise NotImplementedError("write your pallas kernel here")


# speedup vs baseline: 4.6842x; 4.6842x over previous
import jax
import jax.numpy as jnp
from jax import lax
from jax.experimental import pallas as pl
from jax.experimental.pallas import tpu as pltpu
from jax.experimental.pallas import tpu_sc as plsc

_N = 10000
_E = 320000
_G = 64
_EPS = 1e-5

_NC = 2
_NS = 16
_NW = _NC * _NS
_K = 128
_CHN = 80
_EP = _NW * _CHN * _K
_NP = 10240
_TRASH = 10016
_SLAB = _NP // _NS

_mesh = plsc.VectorSubcoreMesh(
    core_axis_name="c", subcore_axis_name="s", num_cores=_NC, num_subcores=_NS
)


def _deg_body(dstH, ewH, z16, out, dst_v, ew_v, msg_v, acc):
    c = lax.axis_index("c")
    s = lax.axis_index("s")
    w = c * _NS + s
    pltpu.sync_copy(z16.at[pl.ds(s * _SLAB, _SLAB)], acc.at[pl.ds(s * _SLAB, _SLAB)])
    pltpu.sync_copy(dstH.at[w], dst_v)
    pltpu.sync_copy(ewH.at[w], ew_v)
    lanes = lax.iota(jnp.int32, 16)
    base0 = jnp.where(lanes == 0, 1.0, 0.0).astype(jnp.float32)
    base1 = jnp.where(lanes == 1, 1.0, 0.0).astype(jnp.float32)
    plsc.subcore_barrier()

    def chunk(j, carry):
        for g in range(_K // 16):
            ew16 = ew_v[j, pl.ds(g * 16, 16)]
            for r in range(16):
                msg_v[g * 16 + r, :] = ew16[r] * base0 + base1
        pltpu.sync_copy(msg_v, acc.at[dst_v.at[j]], add=True)
        return carry

    lax.fori_loop(0, _CHN, chunk, 0)
    plsc.subcore_barrier()
    pltpu.sync_copy(acc.at[pl.ds(s * _SLAB, _SLAB)], out.at[c, pl.ds(s * _SLAB, _SLAB)])


_sc_deg = pl.kernel(
    _deg_body,
    out_type=jax.ShapeDtypeStruct((_NC, _NP, 16), jnp.float32),
    mesh=_mesh,
    scratch_types=[
        pltpu.VMEM((_CHN, _K), jnp.int32),
        pltpu.VMEM((_CHN, _K), jnp.float32),
        pltpu.VMEM((_K, 16), jnp.float32),
        pltpu.VMEM_SHARED((_NP, 16), jnp.float32),
    ],
)


def _pre1_body(xpH, srcH, dstH, ewH, z16, out, src_v, dst_v, ew_v, rows_v,
               msg_v, acc, sem, sidx_v, tab_s):
    c = lax.axis_index("c")
    s = lax.axis_index("s")
    w = c * _NS + s
    pltpu.sync_copy(z16.at[pl.ds(s * _SLAB, _SLAB)], acc.at[pl.ds(s * _SLAB, _SLAB)])
    pltpu.sync_copy(xpH.at[pl.ds(s * _SLAB, _SLAB)], tab_s.at[pl.ds(s * _SLAB, _SLAB)])
    pltpu.sync_copy(srcH.at[w], src_v)
    pltpu.sync_copy(dstH.at[w], dst_v)
    pltpu.sync_copy(ewH.at[w], ew_v)
    plsc.subcore_barrier()

    def chunk(j, carry):
        pltpu.sync_copy(srcH.at[w, j], sidx_v)
        pltpu.sync_copy(tab_s.at[sidx_v], rows_v)
        for g in range(_K // 16):
            ew16 = ew_v[j, pl.ds(g * 16, 16)]
            for r in range(16):
                row0 = rows_v[g * 16 + r, :]
                msg_v[g * 16 + r, :] = row0 * ew16[r]
        pltpu.sync_copy(msg_v, acc.at[dst_v.at[j]], add=True)
        return carry

    lax.fori_loop(0, _CHN, chunk, 0)
    plsc.subcore_barrier()
    pltpu.sync_copy(acc.at[pl.ds(s * _SLAB, _SLAB)], out.at[c, pl.ds(s * _SLAB, _SLAB)])


_sc_pre1 = pl.kernel(
    _pre1_body,
    out_type=jax.ShapeDtypeStruct((_NC, _NP, 16), jnp.float32),
    mesh=_mesh,
    scratch_types=[
        pltpu.VMEM((_CHN, _K), jnp.int32),
        pltpu.VMEM((_CHN, _K), jnp.int32),
        pltpu.VMEM((_CHN, _K), jnp.float32),
        pltpu.VMEM((_K, 16), jnp.float32),
        pltpu.VMEM((_K, 16), jnp.float32),
        pltpu.VMEM_SHARED((_NP, 16), jnp.float32),
        pltpu.SemaphoreType.DMA,
        pltpu.VMEM((_K,), jnp.int32),
        pltpu.VMEM_SHARED((_NP, 16), jnp.float32),
    ],
)


def _big_body(hhH, srcH, dstH, z128, out, src_v, dst_v, rows_v, acc, sem):
    c = lax.axis_index("c")
    s = lax.axis_index("s")
    w = c * _NS + s
    pltpu.sync_copy(z128.at[pl.ds(s * _SLAB, _SLAB)], acc.at[pl.ds(s * _SLAB, _SLAB)])
    pltpu.sync_copy(srcH.at[w], src_v)
    pltpu.sync_copy(dstH.at[w], dst_v)
    plsc.subcore_barrier()

    def chunk(j, carry):
        pltpu.async_copy(hhH.at[src_v.at[j]], rows_v, sem).wait()
        pltpu.sync_copy(rows_v, acc.at[dst_v.at[j]], add=True)
        return carry

    lax.fori_loop(0, _CHN, chunk, 0)
    plsc.subcore_barrier()
    pltpu.sync_copy(acc.at[pl.ds(s * _SLAB, _SLAB)], out.at[c, pl.ds(s * _SLAB, _SLAB)])


_sc_big = pl.kernel(
    _big_body,
    out_type=jax.ShapeDtypeStruct((_NC, _NP, 128), jnp.float32),
    mesh=_mesh,
    scratch_types=[
        pltpu.VMEM((_CHN, _K), jnp.int32),
        pltpu.VMEM((_CHN, _K), jnp.int32),
        pltpu.VMEM((_K, 128), jnp.float32),
        pltpu.VMEM_SHARED((_NP, 128), jnp.float32),
        pltpu.SemaphoreType.DMA,
    ],
)


def _tc1_body(degp_r, x_r, dinv1_o, dinv2_o, xf_o):
    dp = degp_r[...]
    deg1 = dp[0, :_N, 0:1] + dp[1, :_N, 0:1] + 1.0
    deg2 = dp[0, :_N, 1:2] + dp[1, :_N, 1:2] + 1.0
    d1 = jnp.where(deg1 > 0, lax.rsqrt(jnp.maximum(deg1, 1e-12)), 0.0)
    d2 = jnp.where(deg2 > 0, lax.rsqrt(jnp.maximum(deg2, 1e-12)), 0.0)
    dinv1_o[...] = d1
    dinv2_o[...] = d2
    xf_o[0:_N, 0:4] = x_r[...] * d1
    xf_o[0:_N, 4:16] = jnp.zeros((_N, 12), jnp.float32)
    xf_o[_N:_NP, :] = jnp.zeros((_NP - _N, 16), jnp.float32)


def _tc1(degp, x):
    return pl.pallas_call(
        _tc1_body,
        out_shape=[
            jax.ShapeDtypeStruct((_N, 1), jnp.float32),
            jax.ShapeDtypeStruct((_N, 1), jnp.float32),
            jax.ShapeDtypeStruct((_NP, 16), jnp.float32),
        ],
    )(degp, x)


def _tc2_body(pre1_r, x_r, dinv1_r, dinv2_r, W1_r, b1_r, g1_r, be1_r, hh_o):
    pre1 = pre1_r[...]
    d1 = dinv1_r[...]
    agg = d1 * pre1 + (d1 * d1) * x_r[...]
    h = jnp.dot(agg, W1_r[...], preferred_element_type=jnp.float32) + b1_r[...]
    h = jnp.maximum(h, 0.0)
    m = jnp.mean(h, axis=0, keepdims=True)
    cdev = h - m
    v = jnp.mean(cdev * cdev, axis=0, keepdims=True)
    hbn = cdev * lax.rsqrt(v + _EPS) * g1_r[...] + be1_r[...]
    hh_o[...] = hbn * dinv2_r[...]


def _tc2(pre1, x, dinv1, dinv2, W1, b1, g1, be1):
    return pl.pallas_call(
        _tc2_body,
        out_shape=jax.ShapeDtypeStruct((_N, 128), jnp.float32),
    )(pre1, x, dinv1, dinv2, W1, b1, g1, be1)


def _tc3_body(agg2_r, hh_r, dinv2_r, batch_r, W2_r, b2_r, g2_r, be2_r,
              fW1_r, fb1_r, fW2_r, fb2_r, fW3_r, fb3_r, out_o):
    ssum = agg2_r[...] + hh_r[...]
    agg2 = dinv2_r[...] * ssum
    h2 = jnp.dot(agg2, W2_r[...], preferred_element_type=jnp.float32) + b2_r[...]
    h2 = jnp.maximum(h2, 0.0)
    m = jnp.mean(h2, axis=0, keepdims=True)
    cdev = h2 - m
    v = jnp.mean(cdev * cdev, axis=0, keepdims=True)
    h2bn = cdev * lax.rsqrt(v + _EPS) * g2_r[...] + be2_r[...]
    gids = lax.broadcasted_iota(jnp.int32, (_N, _G), 1)
    mask = (batch_r[...] == gids).astype(jnp.float32)
    p = lax.dot_general(mask, h2bn, (((0,), (0,)), ((), ())),
                        preferred_element_type=jnp.float32)
    p = jnp.maximum(jnp.dot(p, fW1_r[...], preferred_element_type=jnp.float32)
                    + fb1_r[...], 0.0)
    p = jnp.maximum(jnp.dot(p, fW2_r[...], preferred_element_type=jnp.float32)
                    + fb2_r[...], 0.0)
    o = jnp.dot(p, fW3_r[...], preferred_element_type=jnp.float32) + fb3_r[...]
    mx = jnp.max(o, axis=1, keepdims=True)
    sh = o - mx
    out_o[...] = sh - jnp.log(jnp.sum(jnp.exp(sh), axis=1, keepdims=True))


def _tc3(agg2, hh, dinv2, batch, W2, b2, g2, be2, fW1, fb1, fW2, fb2, fW3, fb3):
    return pl.pallas_call(
        _tc3_body,
        out_shape=jax.ShapeDtypeStruct((_G, 1), jnp.float32),
    )(agg2, hh, dinv2, batch, W2, b2, g2, be2, fW1, fb1, fW2, fb2, fW3, fb3)


def _tc_dbg_body(a_r, out_o):
    out_o[...] = jnp.sum(a_r[...]) * jnp.zeros((_G, 1), jnp.float32)


def _tc_dbg(a):
    return pl.pallas_call(
        _tc_dbg_body,
        out_shape=jax.ShapeDtypeStruct((_G, 1), jnp.float32),
    )(a)


def kernel(x, edge_index, edge_attr, batch, W1, b1, g1, be1, W2, b2, g2, be2,
           fW1, fb1, fW2, fb2, fW3, fb3):
    src = edge_index[0]
    dst = edge_index[1]
    ew = edge_attr[:, 0]
    pad = _EP - _E
    dst3 = jnp.concatenate(
        [dst, jnp.full((pad,), _TRASH, jnp.int32)]).reshape(_NW, _CHN, _K)
    ew3 = jnp.concatenate(
        [ew, jnp.zeros((pad,), jnp.float32)]).reshape(_NW, _CHN, _K)
    z16 = jnp.zeros((_NP, 16), jnp.float32)

    degp = _sc_deg(dst3, ew3, z16)
    dinv1, dinv2, xf = _tc1(degp, x)

    xs = xf[:_N, 0:4]
    pre1 = jax.ops.segment_sum(xs[src] * ew[:, None], dst, num_segments=_N)
    hh = _tc2(pre1, x, dinv1, dinv2, W1, b1.reshape(1, -1), g1.reshape(1, -1),
              be1.reshape(1, -1))
    agg2 = jax.ops.segment_sum(hh[src], dst, num_segments=_N)
    out = _tc3(agg2, hh, dinv2, batch.reshape(-1, 1), W2, b2.reshape(1, -1),
               g2.reshape(1, -1), be2.reshape(1, -1), fW1, fb1.reshape(1, -1),
               fW2, fb2.reshape(1, -1), fW3, fb3.reshape(1, -1))
    return out
